# Initial kernel scaffold; baseline (speedup 1.0000x reference)
import jax
import jax.numpy as jnp
from jax.experimental import pallas as pl


def kernel(x, table, lora_A, lora_B):
    raise NotImplementedError("write your pallas kernel here")


# speedup vs baseline: 3.3641x; 3.3641x over previous
import functools

import jax
import jax.numpy as jnp
from jax import lax
from jax.experimental import pallas as pl
from jax.experimental.pallas import tpu as pltpu
from jax.experimental.pallas import tpu_sc as plsc

D = 64
R = 16
SCALING = 2.0

NC = 2
NS = 16
NW = NC * NS


def _make_sc_gather(n_idx: int, ch: int):
    bpw = n_idx // NW
    nchunk = bpw // ch
    assert bpw % ch == 0 and n_idx % NW == 0

    mesh = plsc.VectorSubcoreMesh(core_axis_name="c", subcore_axis_name="s")

    @functools.partial(
        pl.kernel,
        mesh=mesh,
        compiler_params=pltpu.CompilerParams(use_tc_tiling_on_sc=False),
        out_type=(
            jax.ShapeDtypeStruct((n_idx, D), jnp.float32),
            jax.ShapeDtypeStruct((n_idx, R), jnp.float32),
        ),
        scratch_types=[
            pltpu.VMEM((ch,), jnp.int32),
            pltpu.VMEM((ch, D), jnp.float32),
            pltpu.VMEM((ch, R), jnp.float32),
            pltpu.SemaphoreType.DMA,
            pltpu.SemaphoreType.DMA,
        ],
    )
    def sc_gather(table_hbm, at_hbm, idx_hbm, base_hbm, aa_hbm,
                  idx_v, rows_v, arows_v, sem1, sem2):
        wid = lax.axis_index("s") * NC + lax.axis_index("c")

        @pl.loop(0, nchunk)
        def _(c):
            off = wid * bpw + c * ch
            pltpu.sync_copy(idx_hbm.at[pl.ds(off, ch)], idx_v)
            cp1 = pltpu.async_copy(table_hbm.at[idx_v], rows_v, sem1)
            cp2 = pltpu.async_copy(at_hbm.at[idx_v], arows_v, sem2)
            cp1.wait()
            cp2.wait()
            pltpu.sync_copy(rows_v, base_hbm.at[pl.ds(off, ch)])
            pltpu.sync_copy(arows_v, aa_hbm.at[pl.ds(off, ch)])

    return sc_gather


def _fuse_body(base_ref, aa_ref, b_ref, o_ref):
    delta = lax.dot_general(
        aa_ref[...], b_ref[...],
        dimension_numbers=(((1,), (1,)), ((), ())),
        preferred_element_type=jnp.float32,
        precision=lax.Precision.HIGHEST,
    )
    o_ref[...] = base_ref[...] + delta * SCALING


def kernel(x, table, lora_A, lora_B):
    bsz, seq = x.shape
    n = bsz * seq
    idx = x.reshape(n).astype(jnp.int32)
    a_t = lora_A.T

    base, aa = _make_sc_gather(n, 640)(table, a_t, idx)

    mblk = 4096
    out = pl.pallas_call(
        _fuse_body,
        grid=(n // mblk,),
        in_specs=[
            pl.BlockSpec((mblk, D), lambda i: (i, 0)),
            pl.BlockSpec((mblk, R), lambda i: (i, 0)),
            pl.BlockSpec((D, R), lambda i: (0, 0)),
        ],
        out_specs=pl.BlockSpec((mblk, D), lambda i: (i, 0)),
        out_shape=jax.ShapeDtypeStruct((n, D), jnp.float32),
    )(base, aa, lora_B)
    return out.reshape(bsz, seq, D)
